# baseline (speedup 1.0000x reference)
import functools
import math

import jax
import jax.numpy as jnp
import numpy as np
from jax import lax
from jax.experimental import pallas as pl
from jax.experimental.pallas import tpu as pltpu
from jax.experimental.pallas import tpu_sc as plsc

VOCAB = 100000
D_MODEL = 768
BATCH = 4
SEQ = 4096
MAX_PERIOD = 10000

LANES = 16
NUM_WORKERS = 32
POS_PER_WORKER = SEQ // NUM_WORKERS
CHUNK = 32
NUM_CHUNKS = POS_PER_WORKER // CHUNK
NUM_STEPS = NUM_CHUNKS * BATCH
VREGS_PER_ROW = D_MODEL // LANES


HALF = D_MODEL // 2
FREQ_GROUPS = HALF // LANES


def _factor_tables():
    freqs = np.exp(
        -math.log(MAX_PERIOD) * np.arange(0, HALF, dtype=np.float64) / HALF
    )
    bases = (np.arange(NUM_WORKERS * NUM_CHUNKS, dtype=np.float64)
             * CHUNK)[:, None] * freqs[None]
    base_tab = np.concatenate([np.cos(bases), np.sin(bases)], axis=-1)
    us = np.arange(CHUNK, dtype=np.float64)[:, None] * freqs[None]
    u_tab = np.concatenate([np.cos(us), np.sin(us)], axis=-1)
    return base_tab.astype(np.float32), u_tab.astype(np.float32)


def kernel(input_ids, token_embedding_table):
    base_tab, u_tab = _factor_tables()

    info = plsc.get_sparse_core_info()
    num_cores = info.num_cores
    mesh = plsc.VectorSubcoreMesh(core_axis_name="c", subcore_axis_name="s")

    @functools.partial(
        pl.kernel,
        mesh=mesh,
        out_type=jax.ShapeDtypeStruct((BATCH * SEQ, D_MODEL), jnp.float32),
        scratch_types=[
            pltpu.VMEM((BATCH, POS_PER_WORKER), jnp.int32),
            pltpu.VMEM((NUM_CHUNKS, D_MODEL), jnp.float32),
            pltpu.VMEM((CHUNK, D_MODEL), jnp.float32),
            pltpu.VMEM((CHUNK, D_MODEL), jnp.float32),
            pltpu.VMEM((CHUNK, D_MODEL), jnp.float32),
            pltpu.VMEM((CHUNK, D_MODEL), jnp.float32),
            pltpu.VMEM((CHUNK, D_MODEL), jnp.float32),
            pltpu.SemaphoreType.DMA,
            pltpu.SemaphoreType.DMA,
            pltpu.SemaphoreType.DMA,
            pltpu.SemaphoreType.DMA,
            pltpu.SemaphoreType.DMA,
            pltpu.SemaphoreType.DMA,
            pltpu.SemaphoreType.DMA,
            pltpu.SemaphoreType.DMA,
        ],
    )
    def embed(ids_hbm, table_hbm, base_hbm, u_hbm, out_hbm,
              idx_all, bbuf, ubuf, poschunk, rows0, rows1, rows2,
              bsem, usem, g0, g1, g2, w0, w1, w2):
        wid = lax.axis_index("s") * num_cores + lax.axis_index("c")
        s0 = wid * POS_PER_WORKER
        rows = (rows0, rows1, rows2)
        gsem = (g0, g1, g2)
        wsem = (w0, w1, w2)

        basecp = pltpu.async_copy(
            base_hbm.at[pl.ds(wid * NUM_CHUNKS, NUM_CHUNKS)], bbuf, bsem)
        ucp = pltpu.async_copy(u_hbm, ubuf, usem)
        for b in range(BATCH):
            pltpu.sync_copy(ids_hbm.at[b, pl.ds(s0, POS_PER_WORKER)],
                            idx_all.at[b])

        def issue_gather(t):
            c, b, j = t // BATCH, t % BATCH, t % 3
            return pltpu.async_copy(
                table_hbm.at[idx_all.at[b, pl.ds(c * CHUNK, CHUNK)]],
                rows[j], gsem[j])

        gcopies = [None] * NUM_STEPS
        wcopies = [None] * NUM_STEPS
        gcopies[0] = issue_gather(0)
        gcopies[1] = issue_gather(1)
        basecp.wait()
        ucp.wait()

        for t in range(NUM_STEPS):
            c, b, j = t // BATCH, t % BATCH, t % 3
            if b == 0:
                for half_pass in range(2):
                    gs = range(half_pass * FREQ_GROUPS // 2,
                               (half_pass + 1) * FREQ_GROUPS // 2)
                    slices = [(pl.ds(g * LANES, LANES),
                               pl.ds(HALF + g * LANES, LANES)) for g in gs]
                    factors = [(bbuf[c, cg], bbuf[c, sg])
                               for cg, sg in slices]

                    def mat_r(r, rcarry, slices=slices, factors=factors):
                        for (cg, sg), (bc, bs) in zip(slices, factors):
                            uc = ubuf[r, cg]
                            us = ubuf[r, sg]
                            poschunk[r, cg] = bc * uc - bs * us
                            poschunk[r, sg] = bs * uc + bc * us
                        return rcarry

                    lax.fori_loop(0, CHUNK, mat_r, 0)
            if t + 2 < NUM_STEPS:
                if t >= 1:
                    wcopies[t - 1].wait()
                gcopies[t + 2] = issue_gather(t + 2)
            gcopies[t].wait()
            rv = rows[j]
            pv = poschunk

            def row_add(r, carry):
                for g in range(VREGS_PER_ROW):
                    sl = pl.ds(g * LANES, LANES)
                    plsc.addupdate(rv.at[r, sl], pv[r, sl])
                return carry

            lax.fori_loop(0, CHUNK, row_add, 0)
            wcopies[t] = pltpu.async_copy(
                rv, out_hbm.at[pl.ds(b * SEQ + s0 + c * CHUNK, CHUNK)], wsem[j])
        wcopies[NUM_STEPS - 3].wait()
        wcopies[NUM_STEPS - 2].wait()
        wcopies[NUM_STEPS - 1].wait()

    out = embed(input_ids, token_embedding_table, base_tab, u_tab)
    return out.reshape(BATCH, SEQ, D_MODEL)


# speedup vs baseline: 1.0815x; 1.0815x over previous
import functools
import math

import jax
import jax.numpy as jnp
import numpy as np
from jax import lax
from jax.experimental import pallas as pl
from jax.experimental.pallas import tpu as pltpu
from jax.experimental.pallas import tpu_sc as plsc

VOCAB = 100000
D_MODEL = 768
BATCH = 4
SEQ = 4096
MAX_PERIOD = 10000

LANES = 16
NUM_WORKERS = 32
POS_PER_WORKER = SEQ // NUM_WORKERS
CHUNK = 32
NUM_CHUNKS = POS_PER_WORKER // CHUNK
NUM_STEPS = NUM_CHUNKS * BATCH
VREGS_PER_ROW = D_MODEL // LANES


HALF = D_MODEL // 2
FREQ_GROUPS = HALF // LANES


def _factor_tables():
    freqs = np.exp(
        -math.log(MAX_PERIOD) * np.arange(0, HALF, dtype=np.float64) / HALF
    )
    bases = (np.arange(NUM_WORKERS * NUM_CHUNKS, dtype=np.float64)
             * CHUNK)[:, None] * freqs[None]
    base_tab = np.concatenate([np.cos(bases), np.sin(bases)], axis=-1)
    us = np.arange(CHUNK, dtype=np.float64)[:, None] * freqs[None]
    u_tab = np.concatenate([np.cos(us), np.sin(us)], axis=-1)
    return base_tab.astype(np.float32), u_tab.astype(np.float32)


def kernel(input_ids, token_embedding_table):
    base_tab, u_tab = _factor_tables()

    info = plsc.get_sparse_core_info()
    num_cores = info.num_cores
    mesh = plsc.VectorSubcoreMesh(core_axis_name="c", subcore_axis_name="s")

    @functools.partial(
        pl.kernel,
        mesh=mesh,
        out_type=jax.ShapeDtypeStruct((BATCH * SEQ, D_MODEL), jnp.float32),
        scratch_types=[
            pltpu.VMEM((BATCH, POS_PER_WORKER), jnp.int32),
            pltpu.VMEM((NUM_CHUNKS, D_MODEL), jnp.float32),
            pltpu.VMEM((CHUNK, D_MODEL), jnp.float32),
            pltpu.VMEM((CHUNK, D_MODEL), jnp.float32),
            pltpu.VMEM((CHUNK, D_MODEL), jnp.float32),
            pltpu.VMEM((CHUNK, D_MODEL), jnp.float32),
            pltpu.VMEM((CHUNK, D_MODEL), jnp.float32),
            pltpu.SemaphoreType.DMA,
            pltpu.SemaphoreType.DMA,
            pltpu.SemaphoreType.DMA,
            pltpu.SemaphoreType.DMA,
            pltpu.SemaphoreType.DMA,
            pltpu.SemaphoreType.DMA,
            pltpu.SemaphoreType.DMA,
            pltpu.SemaphoreType.DMA,
        ],
    )
    def embed(ids_hbm, table_hbm, base_hbm, u_hbm, out_hbm,
              idx_all, bbuf, ubuf, poschunk, rows0, rows1, rows2,
              bsem, usem, g0, g1, g2, w0, w1, w2):
        wid = lax.axis_index("s") * num_cores + lax.axis_index("c")
        s0 = wid * POS_PER_WORKER
        rows = (rows0, rows1, rows2)
        gsem = (g0, g1, g2)
        wsem = (w0, w1, w2)

        basecp = pltpu.async_copy(
            base_hbm.at[pl.ds(wid * NUM_CHUNKS, NUM_CHUNKS)], bbuf, bsem)
        ucp = pltpu.async_copy(u_hbm, ubuf, usem)
        for b in range(BATCH):
            pltpu.sync_copy(ids_hbm.at[b, pl.ds(s0, POS_PER_WORKER)],
                            idx_all.at[b])

        def issue_gather(t):
            c, b, j = t // BATCH, t % BATCH, t % 3
            return pltpu.async_copy(
                table_hbm.at[idx_all.at[b, pl.ds(c * CHUNK, CHUNK)]],
                rows[j], gsem[j])

        gcopies = [None] * NUM_STEPS
        wcopies = [None] * NUM_STEPS
        gcopies[0] = issue_gather(0)
        gcopies[1] = issue_gather(1)
        basecp.wait()
        ucp.wait()

        for t in range(NUM_STEPS):
            c, b, j = t // BATCH, t % BATCH, t % 3
            if b == 0:
                for half_pass in range(2):
                    gs = range(half_pass * FREQ_GROUPS // 2,
                               (half_pass + 1) * FREQ_GROUPS // 2)
                    slices = [(pl.ds(g * LANES, LANES),
                               pl.ds(HALF + g * LANES, LANES)) for g in gs]
                    factors = [(bbuf[c, cg], bbuf[c, sg])
                               for cg, sg in slices]

                    def mat_r(r, rcarry, slices=slices, factors=factors):
                        for (cg, sg), (bc, bs) in zip(slices, factors):
                            uc = ubuf[r, cg]
                            us = ubuf[r, sg]
                            poschunk[r, cg] = bc * uc - bs * us
                            poschunk[r, sg] = bs * uc + bc * us
                        return rcarry

                    lax.fori_loop(0, CHUNK, mat_r, 0)
            gcopies[t].wait()
            rv = rows[j]
            pv = poschunk

            def row_add(r, carry):
                for g in range(VREGS_PER_ROW):
                    sl = pl.ds(g * LANES, LANES)
                    plsc.addupdate(rv.at[r, sl], pv[r, sl])
                return carry

            lax.fori_loop(0, CHUNK, row_add, 0)
            wcopies[t] = pltpu.async_copy(
                rv, out_hbm.at[pl.ds(b * SEQ + s0 + c * CHUNK, CHUNK)], wsem[j])
            if t + 2 < NUM_STEPS:
                if t >= 1:
                    wcopies[t - 1].wait()
                gcopies[t + 2] = issue_gather(t + 2)
        wcopies[NUM_STEPS - 3].wait()
        wcopies[NUM_STEPS - 2].wait()
        wcopies[NUM_STEPS - 1].wait()

    out = embed(input_ids, token_embedding_table, base_tab, u_tab)
    return out.reshape(BATCH, SEQ, D_MODEL)
